# baseline (speedup 1.0000x reference)
import functools

import numpy as np
import jax
import jax.numpy as jnp
from jax import lax
from jax.experimental import pallas as pl
from jax.experimental.pallas import tpu as pltpu
from jax.experimental.pallas import tpu_sc as plsc

_B = 10
_H = 400
_MAIN_IN = 128
_MAIN_OUT = 64
_SEG_EA = [_MAIN_IN * _H, _H, _H * _H, _H, _H * _H, _H, _H * _MAIN_OUT, _MAIN_OUT]
_SEG_START = [0, 51200, 51600, 211600, 212000, 372000, 372400, 398000]
_SEG_SHAPES = [
    (_B, _H, _MAIN_IN), (_B, _H), (_B, _H, _H), (_B, _H),
    (_B, _H, _H), (_B, _H), (_B, _MAIN_OUT, _H), (_B, _MAIN_OUT),
]
_TOTAL = sum(_SEG_EA)
_NSEG = len(_SEG_EA)
_BIG = [0, 2, 4, 6]
_SMALL = [1, 3, 5, 7]

_W = 8192
_NSTEPS = -(-_TOTAL // _W)

_WINW = 8192
_INT_MIN = -2147483648


def _gumbel_noise() -> jax.Array:
    base = jax.random.key(42)
    parts = [
        jax.random.gumbel(jax.random.fold_in(base, i), (_B, ea), jnp.float32)
        for i, ea in enumerate(_SEG_EA)
    ]
    return jnp.concatenate(parts, axis=1)


def _mm_body(emb_ref, g1w_ref, g1b_ref, g2w_ref, g2b_ref, g_ref, u_ref):
    h = jnp.maximum(
        lax.dot_general(emb_ref[...], g1w_ref[...], (((1,), (1,)), ((), ())),
                        preferred_element_type=jnp.float32) + g1b_ref[...],
        0.0)
    z = (lax.dot_general(h, g2w_ref[...], (((1,), (1,)), ((), ())),
                         preferred_element_type=jnp.float32)
         + g2b_ref[...] + g_ref[...])
    zi = lax.bitcast_convert_type(z, jnp.int32)
    u_ref[...] = zi ^ ((zi >> 31) | jnp.int32(_INT_MIN))


def _mm_call(emb, g1w, g1b, g2w, g2b, gnoise):
    return pl.pallas_call(
        _mm_body,
        grid=(_NSTEPS,),
        in_specs=[
            pl.BlockSpec((_B, 10), lambda i: (0, 0)),
            pl.BlockSpec((256, 10), lambda i: (0, 0)),
            pl.BlockSpec((1, 256), lambda i: (0, 0)),
            pl.BlockSpec((_W, 256), lambda i: (i, 0)),
            pl.BlockSpec((1, _W), lambda i: (0, i)),
            pl.BlockSpec((_B, _W), lambda i: (0, i)),
        ],
        out_specs=pl.BlockSpec((_B, _W), lambda i: (0, i)),
        out_shape=jax.ShapeDtypeStruct((_B, _TOTAL), jnp.int32),
    )(emb, g1w, g1b, g2w, g2b, gnoise)


_PHASES = [(12, 20, None), (12, 8, 20), (8, 0, 8)]


def _sc_body(u_hbm, thr_hbm, win, hist, merged, thrv):
    w = lax.axis_index("s") * 2 + lax.axis_index("c")
    lane = lax.iota(jnp.int32, 16)
    ones = jnp.ones((16,), jnp.int32)

    def clear(nbins):
        def zi(i, c):
            hist[pl.ds(i * 16, 16)] = jnp.zeros((16,), jnp.int32)
            return c
        lax.fori_loop(0, nbins, zi, 0)

    def scan(off, n, nbins, shift, prefshift, prefix):
        lbase = lane * nbins

        def do_vreg(vi, c):
            v = win[pl.ds(vi * 16, 16)]
            bkt = lax.shift_right_logical(v, shift) & jnp.int32(nbins - 1)
            idx = lbase + bkt
            if prefshift is None:
                plsc.addupdate_scatter(hist, [idx], ones)
            else:
                m = lax.shift_right_logical(v, prefshift) == prefix
                plsc.addupdate_scatter(hist, [idx], ones, mask=m)
            return c

        nfull = n // _WINW
        rem = n - nfull * _WINW

        def win_loop(b, c):
            pltpu.sync_copy(u_hbm.at[pl.ds(off + b * _WINW, _WINW)], win)
            lax.fori_loop(0, _WINW // 16, do_vreg, 0)
            return c
        lax.fori_loop(0, nfull, win_loop, 0)
        if rem:
            pltpu.sync_copy(u_hbm.at[pl.ds(off + nfull * _WINW, _WINW)], win)
            lax.fori_loop(0, rem // 16, do_vreg, 0)

    def merge_and_find(nbins, krem):
        ngr = nbins // 16

        def mg(g, c):
            acc = hist[pl.ds(g * 16, 16)]
            for l in range(1, 16):
                acc = acc + hist[pl.ds(l * nbins + g * 16, 16)]
            merged[pl.ds(g * 16, 16)] = acc
            return c
        lax.fori_loop(0, ngr, mg, 0)

        def gs(i, carry):
            cum, fg, cb = carry
            g = ngr - 1 - i
            sg = jnp.sum(merged[pl.ds(g * 16, 16)])
            newcum = cum + sg
            hit = jnp.logical_and(cum < krem, newcum >= krem)
            return (newcum,
                    jnp.where(hit, g, fg),
                    jnp.where(hit, cum, cb))
        _, fg, cb = lax.fori_loop(
            0, ngr, gs, (jnp.int32(0), jnp.int32(0), jnp.int32(0)))

        v = merged[pl.ds(fg * 16, 16)]
        csum = plsc.cumsum(v)
        tot = jnp.sum(v)
        above = cb + (tot - csum)
        hitl = jnp.logical_and(above < krem, above + v >= krem)
        li = jnp.sum(jnp.where(hitl, lane, jnp.int32(0)))
        ab = jnp.sum(jnp.where(hitl, above, jnp.int32(0)))
        return fg * 16 + li, krem - ab

    def run_task(off, n, k):
        prefix = jnp.int32(0)
        krem = jnp.int32(k)
        for nbits, shift, prefshift in _PHASES:
            nbins = 1 << nbits
            clear(nbins)
            scan(off, n, nbins, shift, prefshift, prefix)
            bstar, krem = merge_and_find(nbins, krem)
            prefix = (prefix << nbits) | bstar
        return prefix ^ jnp.int32(_INT_MIN)

    T = _TOTAL

    @pl.when(w < 20)
    def _():
        row = jnp.where(w < 10, w, w - 10)
        segoff = jnp.where(w < 10, jnp.int32(_SEG_START[2]),
                           jnp.int32(_SEG_START[4]))
        thrv[pl.ds(0, 16)] = jnp.broadcast_to(
            run_task(row * T + segoff, 160000, 80000), (16,))

    @pl.when(w >= 20)
    def _():
        r = w - 20
        t0 = run_task(r * T + _SEG_START[0], 51200, 25600)
        t1 = run_task(r * T + _SEG_START[6], 25600, 12800)
        thrv[pl.ds(0, 16)] = jnp.where(lane == 1, t1, t0)

    pltpu.sync_copy(thrv, thr_hbm.at[w])


def _sc_thresholds(u_flat):
    mesh = plsc.VectorSubcoreMesh(core_axis_name="c", subcore_axis_name="s")
    f = pl.kernel(
        _sc_body,
        mesh=mesh,
        out_type=jax.ShapeDtypeStruct((32, 16), jnp.int32),
        scratch_types=[
            pltpu.VMEM((_WINW,), jnp.int32),
            pltpu.VMEM((16 * 4096,), jnp.int32),
            pltpu.VMEM((4096,), jnp.int32),
            pltpu.VMEM((16,), jnp.int32),
        ],
        compiler_params=pltpu.CompilerParams(needs_layout_passes=False),
    )
    return f(u_flat)


def _mask_body(u_ref, thr_ref, *out_refs):
    for t, j in enumerate(_BIG):
        si, ea = _SEG_START[j], _SEG_EA[j]
        s = u_ref[:, si:si + ea] ^ jnp.int32(_INT_MIN)
        out_refs[j][...] = (s >= thr_ref[:, t:t + 1]).astype(jnp.float32)
    for j in _SMALL:
        si, ea = _SEG_START[j], _SEG_EA[j]
        k = ea // 2
        s = u_ref[:, si:si + ea] ^ jnp.int32(_INT_MIN)
        cnt0 = jnp.sum((s >= 0).astype(jnp.int32), axis=1, keepdims=True)
        t0 = jnp.where(cnt0 >= k, jnp.int32(0), jnp.int32(_INT_MIN))

        def body(t, thr, s=s, k=k):
            cand = thr | (jnp.int32(1) << (30 - t))
            cnt = jnp.sum((s >= cand).astype(jnp.int32), axis=1, keepdims=True)
            return jnp.where(cnt >= k, cand, thr)

        thr = lax.fori_loop(0, 31, body, t0)
        out_refs[j][...] = (s >= thr).astype(jnp.float32)


def _mask_call(u, thr4):
    return pl.pallas_call(
        _mask_body,
        out_shape=[jax.ShapeDtypeStruct((_B, ea), jnp.float32) for ea in _SEG_EA],
        compiler_params=pltpu.CompilerParams(vmem_limit_bytes=100 * 1024 * 1024),
    )(u, thr4)


def kernel(x, embedding_input, g1_w, g1_b, g2_w, g2_b):
    del x
    u = _mm_call(embedding_input, g1_w, g1_b.reshape(1, 256), g2_w,
                 g2_b.reshape(1, _TOTAL), _gumbel_noise())
    u_flat = jnp.concatenate([u.reshape(-1), jnp.zeros((_WINW,), jnp.int32)])
    thr = _sc_thresholds(u_flat)
    thr4 = jnp.stack([thr[20:30, 0], thr[0:10, 0], thr[10:20, 0],
                      thr[20:30, 1]], axis=1)
    ms = _mask_call(u, thr4)
    return tuple(m.reshape(shp) for m, shp in zip(ms, _SEG_SHAPES))


# speedup vs baseline: 1.0646x; 1.0646x over previous
import functools

import numpy as np
import jax
import jax.numpy as jnp
from jax import lax
from jax.experimental import pallas as pl
from jax.experimental.pallas import tpu as pltpu
from jax.experimental.pallas import tpu_sc as plsc

_B = 10
_H = 400
_MAIN_IN = 128
_MAIN_OUT = 64
_SEG_EA = [_MAIN_IN * _H, _H, _H * _H, _H, _H * _H, _H, _H * _MAIN_OUT, _MAIN_OUT]
_SEG_START = [0, 51200, 51600, 211600, 212000, 372000, 372400, 398000]
_SEG_SHAPES = [
    (_B, _H, _MAIN_IN), (_B, _H), (_B, _H, _H), (_B, _H),
    (_B, _H, _H), (_B, _H), (_B, _MAIN_OUT, _H), (_B, _MAIN_OUT),
]
_TOTAL = sum(_SEG_EA)
_NSEG = len(_SEG_EA)
_BIG = [0, 2, 4, 6]
_SMALL = [1, 3, 5, 7]

_W = 8192
_NSTEPS = -(-_TOTAL // _W)

_WINW = 16384
_INT_MIN = -2147483648


def _gumbel_noise() -> jax.Array:
    base = jax.random.key(42)
    parts = [
        jax.random.gumbel(jax.random.fold_in(base, i), (_B, ea), jnp.float32)
        for i, ea in enumerate(_SEG_EA)
    ]
    return jnp.concatenate(parts, axis=1)


def _mm_body(emb_ref, g1w_ref, g1b_ref, g2w_ref, g2b_ref, g_ref, u_ref):
    h = jnp.maximum(
        lax.dot_general(emb_ref[...], g1w_ref[...], (((1,), (1,)), ((), ())),
                        preferred_element_type=jnp.float32) + g1b_ref[...],
        0.0)
    z = (lax.dot_general(h, g2w_ref[...], (((1,), (1,)), ((), ())),
                         preferred_element_type=jnp.float32)
         + g2b_ref[...] + g_ref[...])
    zi = lax.bitcast_convert_type(z, jnp.int32)
    u_ref[...] = zi ^ ((zi >> 31) | jnp.int32(_INT_MIN))


def _mm_call(emb, g1w, g1b, g2w, g2b, gnoise):
    return pl.pallas_call(
        _mm_body,
        grid=(_NSTEPS,),
        in_specs=[
            pl.BlockSpec((_B, 10), lambda i: (0, 0)),
            pl.BlockSpec((256, 10), lambda i: (0, 0)),
            pl.BlockSpec((1, 256), lambda i: (0, 0)),
            pl.BlockSpec((_W, 256), lambda i: (i, 0)),
            pl.BlockSpec((1, _W), lambda i: (0, i)),
            pl.BlockSpec((_B, _W), lambda i: (0, i)),
        ],
        out_specs=pl.BlockSpec((_B, _W), lambda i: (0, i)),
        out_shape=jax.ShapeDtypeStruct((_B, _TOTAL), jnp.int32),
    )(emb, g1w, g1b, g2w, g2b, gnoise)


_PHASES = [(12, 20, None), (12, 8, 20), (8, 0, 8)]


def _sc_body(u_hbm, thr_hbm, win_a, win_b, hist, merged, thrv, sem_a, sem_b):
    w = lax.axis_index("s") * 2 + lax.axis_index("c")
    lane = lax.iota(jnp.int32, 16)
    ones = jnp.ones((16,), jnp.int32)

    def clear(nbins):
        def zi(i, c):
            hist[pl.ds(i * 16, 16)] = jnp.zeros((16,), jnp.int32)
            return c
        lax.fori_loop(0, nbins, zi, 0)

    def scan(off, n, nbins, shift, prefshift, prefix):
        lbase = lane * nbins

        def do_vreg(vref, vi, c):
            v = vref[pl.ds(vi * 16, 16)]
            bkt = lax.shift_right_logical(v, shift) & jnp.int32(nbins - 1)
            idx = lbase + bkt
            if prefshift is None:
                plsc.addupdate_scatter(hist, [idx], ones)
            else:
                m = lax.shift_right_logical(v, prefshift) == prefix
                plsc.addupdate_scatter(hist, [idx], ones, mask=m)
            return c

        bufs, sems = (win_a, win_b), (sem_a, sem_b)
        nwin = -(-n // _WINW)
        cps = [None, None]
        cps[0] = pltpu.async_copy(u_hbm.at[pl.ds(off, _WINW)], bufs[0], sems[0])
        for b in range(nwin):
            cur = b % 2
            if b + 1 < nwin:
                nxt = (b + 1) % 2
                cps[nxt] = pltpu.async_copy(
                    u_hbm.at[pl.ds(off + (b + 1) * _WINW, _WINW)],
                    bufs[nxt], sems[nxt])
            cps[cur].wait()
            m = min(_WINW, n - b * _WINW)
            lax.fori_loop(0, m // 16, functools.partial(do_vreg, bufs[cur]), 0)

    def merge_and_find(nbins, krem):
        ngr = nbins // 16

        def mg(g, c):
            acc = hist[pl.ds(g * 16, 16)]
            for l in range(1, 16):
                acc = acc + hist[pl.ds(l * nbins + g * 16, 16)]
            merged[pl.ds(g * 16, 16)] = acc
            return c
        lax.fori_loop(0, ngr, mg, 0)

        def gs(i, carry):
            cum, fg, cb = carry
            g = ngr - 1 - i
            sg = jnp.sum(merged[pl.ds(g * 16, 16)])
            newcum = cum + sg
            hit = jnp.logical_and(cum < krem, newcum >= krem)
            return (newcum,
                    jnp.where(hit, g, fg),
                    jnp.where(hit, cum, cb))
        _, fg, cb = lax.fori_loop(
            0, ngr, gs, (jnp.int32(0), jnp.int32(0), jnp.int32(0)))

        v = merged[pl.ds(fg * 16, 16)]
        csum = plsc.cumsum(v)
        tot = jnp.sum(v)
        above = cb + (tot - csum)
        hitl = jnp.logical_and(above < krem, above + v >= krem)
        li = jnp.sum(jnp.where(hitl, lane, jnp.int32(0)))
        ab = jnp.sum(jnp.where(hitl, above, jnp.int32(0)))
        return fg * 16 + li, krem - ab

    def run_task(off, n, k):
        prefix = jnp.int32(0)
        krem = jnp.int32(k)
        for nbits, shift, prefshift in _PHASES:
            nbins = 1 << nbits
            clear(nbins)
            scan(off, n, nbins, shift, prefshift, prefix)
            bstar, krem = merge_and_find(nbins, krem)
            prefix = (prefix << nbits) | bstar
        return prefix ^ jnp.int32(_INT_MIN)

    T = _TOTAL

    @pl.when(w < 20)
    def _():
        row = jnp.where(w < 10, w, w - 10)
        segoff = jnp.where(w < 10, jnp.int32(_SEG_START[2]),
                           jnp.int32(_SEG_START[4]))
        thrv[pl.ds(0, 16)] = jnp.broadcast_to(
            run_task(row * T + segoff, 160000, 80000), (16,))

    @pl.when(w >= 20)
    def _():
        r = w - 20
        t0 = run_task(r * T + _SEG_START[0], 51200, 25600)
        t1 = run_task(r * T + _SEG_START[6], 25600, 12800)
        thrv[pl.ds(0, 16)] = jnp.where(lane == 1, t1, t0)

    pltpu.sync_copy(thrv, thr_hbm.at[w])


def _sc_thresholds(u_flat):
    mesh = plsc.VectorSubcoreMesh(core_axis_name="c", subcore_axis_name="s")
    f = pl.kernel(
        _sc_body,
        mesh=mesh,
        out_type=jax.ShapeDtypeStruct((32, 16), jnp.int32),
        scratch_types=[
            pltpu.VMEM((_WINW,), jnp.int32),
            pltpu.VMEM((_WINW,), jnp.int32),
            pltpu.VMEM((16 * 4096,), jnp.int32),
            pltpu.VMEM((4096,), jnp.int32),
            pltpu.VMEM((16,), jnp.int32),
            pltpu.SemaphoreType.DMA,
            pltpu.SemaphoreType.DMA,
        ],
        compiler_params=pltpu.CompilerParams(needs_layout_passes=False),
    )
    return f(u_flat)


def _mask_body(u_ref, thr_ref, *out_refs):
    for t, j in enumerate(_BIG):
        si, ea = _SEG_START[j], _SEG_EA[j]
        s = u_ref[:, si:si + ea] ^ jnp.int32(_INT_MIN)
        out_refs[j][...] = (s >= thr_ref[:, t:t + 1]).astype(jnp.float32)
    for j in _SMALL:
        si, ea = _SEG_START[j], _SEG_EA[j]
        k = ea // 2
        s = u_ref[:, si:si + ea] ^ jnp.int32(_INT_MIN)
        cnt0 = jnp.sum((s >= 0).astype(jnp.int32), axis=1, keepdims=True)
        t0 = jnp.where(cnt0 >= k, jnp.int32(0), jnp.int32(_INT_MIN))

        def body(t, thr, s=s, k=k):
            cand = thr | (jnp.int32(1) << (30 - t))
            cnt = jnp.sum((s >= cand).astype(jnp.int32), axis=1, keepdims=True)
            return jnp.where(cnt >= k, cand, thr)

        thr = lax.fori_loop(0, 31, body, t0)
        out_refs[j][...] = (s >= thr).astype(jnp.float32)


def _mask_call(u, thr4):
    return pl.pallas_call(
        _mask_body,
        out_shape=[jax.ShapeDtypeStruct((_B, ea), jnp.float32) for ea in _SEG_EA],
        compiler_params=pltpu.CompilerParams(vmem_limit_bytes=100 * 1024 * 1024),
    )(u, thr4)


def kernel(x, embedding_input, g1_w, g1_b, g2_w, g2_b):
    del x
    u = _mm_call(embedding_input, g1_w, g1_b.reshape(1, 256), g2_w,
                 g2_b.reshape(1, _TOTAL), _gumbel_noise())
    u_flat = jnp.concatenate([u.reshape(-1), jnp.zeros((_WINW,), jnp.int32)])
    thr = _sc_thresholds(u_flat)
    thr4 = jnp.stack([thr[20:30, 0], thr[0:10, 0], thr[10:20, 0],
                      thr[20:30, 1]], axis=1)
    ms = _mask_call(u, thr4)
    return tuple(m.reshape(shp) for m, shp in zip(ms, _SEG_SHAPES))


# speedup vs baseline: 1.0864x; 1.0205x over previous
import functools

import numpy as np
import jax
import jax.numpy as jnp
from jax import lax
from jax.experimental import pallas as pl
from jax.experimental.pallas import tpu as pltpu
from jax.experimental.pallas import tpu_sc as plsc

_B = 10
_H = 400
_MAIN_IN = 128
_MAIN_OUT = 64
_SEG_EA = [_MAIN_IN * _H, _H, _H * _H, _H, _H * _H, _H, _H * _MAIN_OUT, _MAIN_OUT]
_SEG_START = [0, 51200, 51600, 211600, 212000, 372000, 372400, 398000]
_SEG_SHAPES = [
    (_B, _H, _MAIN_IN), (_B, _H), (_B, _H, _H), (_B, _H),
    (_B, _H, _H), (_B, _H), (_B, _MAIN_OUT, _H), (_B, _MAIN_OUT),
]
_TOTAL = sum(_SEG_EA)
_NSEG = len(_SEG_EA)
_BIG = [0, 2, 4, 6]
_SMALL = [1, 3, 5, 7]

_W = 8192
_NSTEPS = -(-_TOTAL // _W)

_WINW = 16384
_INT_MIN = -2147483648


def _gumbel_noise() -> jax.Array:
    base = jax.random.key(42)
    parts = [
        jax.random.gumbel(jax.random.fold_in(base, i), (_B, ea), jnp.float32)
        for i, ea in enumerate(_SEG_EA)
    ]
    return jnp.concatenate(parts, axis=1)


def _mm_body(emb_ref, g1w_ref, g1b_ref, g2w_ref, g2b_ref, g_ref, u_ref):
    h = jnp.maximum(
        lax.dot_general(emb_ref[...], g1w_ref[...], (((1,), (1,)), ((), ())),
                        preferred_element_type=jnp.float32) + g1b_ref[...],
        0.0)
    z = (lax.dot_general(h, g2w_ref[...], (((1,), (1,)), ((), ())),
                         preferred_element_type=jnp.float32)
         + g2b_ref[...] + g_ref[...])
    zi = lax.bitcast_convert_type(z, jnp.int32)
    u_ref[...] = zi ^ ((zi >> 31) | jnp.int32(_INT_MIN))


def _mm_call(emb, g1w, g1b, g2w, g2b, gnoise):
    return pl.pallas_call(
        _mm_body,
        grid=(_NSTEPS,),
        in_specs=[
            pl.BlockSpec((_B, 10), lambda i: (0, 0)),
            pl.BlockSpec((256, 10), lambda i: (0, 0)),
            pl.BlockSpec((1, 256), lambda i: (0, 0)),
            pl.BlockSpec((_W, 256), lambda i: (i, 0)),
            pl.BlockSpec((1, _W), lambda i: (0, i)),
            pl.BlockSpec((_B, _W), lambda i: (0, i)),
        ],
        out_specs=pl.BlockSpec((_B, _W), lambda i: (0, i)),
        out_shape=jax.ShapeDtypeStruct((_B, _TOTAL), jnp.int32),
    )(emb, g1w, g1b, g2w, g2b, gnoise)


_PHASES = [(12, 20, None), (12, 8, 20), (8, 0, 8)]


def _sc_body(u_hbm, thr_hbm, win_a, win_b, hist, merged, thrv, sem_a, sem_b):
    w = lax.axis_index("s") * 2 + lax.axis_index("c")
    lane = lax.iota(jnp.int32, 16)
    ones = jnp.ones((16,), jnp.int32)

    def clear(nbins):
        def zi(i, c):
            hist[pl.ds(i * 16, 16)] = jnp.zeros((16,), jnp.int32)
            return c
        lax.fori_loop(0, nbins, zi, 0)

    def scan(off, n, nbins, shift, prefshift, prefix):
        lbase = lane * nbins

        def do_vreg(vref, vi, c):
            v = vref[pl.ds(vi * 16, 16)]
            bkt = lax.shift_right_logical(v, shift) & jnp.int32(nbins - 1)
            idx = lbase + bkt
            if prefshift is None:
                plsc.addupdate_scatter(hist, [idx], ones)
            else:
                m = lax.shift_right_logical(v, prefshift) == prefix
                plsc.addupdate_scatter(hist, [idx], ones, mask=m)
            return c

        bufs, sems = (win_a, win_b), (sem_a, sem_b)
        nwin = -(-n // _WINW)
        cps = [None, None]
        cps[0] = pltpu.async_copy(u_hbm.at[pl.ds(off, _WINW)], bufs[0], sems[0])
        for b in range(nwin):
            cur = b % 2
            if b + 1 < nwin:
                nxt = (b + 1) % 2
                cps[nxt] = pltpu.async_copy(
                    u_hbm.at[pl.ds(off + (b + 1) * _WINW, _WINW)],
                    bufs[nxt], sems[nxt])
            cps[cur].wait()
            m = min(_WINW, n - b * _WINW)

            def quad(q, c, vref=bufs[cur]):
                for dq in range(4):
                    do_vreg(vref, q * 4 + dq, c)
                return c
            lax.fori_loop(0, m // 64, quad, 0)

    def merge_and_find(nbins, krem):
        ngr = nbins // 16

        def mg(g, c):
            acc = hist[pl.ds(g * 16, 16)]
            for l in range(1, 16):
                acc = acc + hist[pl.ds(l * nbins + g * 16, 16)]
            merged[pl.ds(g * 16, 16)] = acc
            return c
        lax.fori_loop(0, ngr, mg, 0)

        def gs(i, carry):
            cum, fg, cb = carry
            g = ngr - 1 - i
            sg = jnp.sum(merged[pl.ds(g * 16, 16)])
            newcum = cum + sg
            hit = jnp.logical_and(cum < krem, newcum >= krem)
            return (newcum,
                    jnp.where(hit, g, fg),
                    jnp.where(hit, cum, cb))
        _, fg, cb = lax.fori_loop(
            0, ngr, gs, (jnp.int32(0), jnp.int32(0), jnp.int32(0)))

        v = merged[pl.ds(fg * 16, 16)]
        csum = plsc.cumsum(v)
        tot = jnp.sum(v)
        above = cb + (tot - csum)
        hitl = jnp.logical_and(above < krem, above + v >= krem)
        li = jnp.sum(jnp.where(hitl, lane, jnp.int32(0)))
        ab = jnp.sum(jnp.where(hitl, above, jnp.int32(0)))
        return fg * 16 + li, krem - ab

    def run_task(off, n, k):
        prefix = jnp.int32(0)
        krem = jnp.int32(k)
        for nbits, shift, prefshift in _PHASES:
            nbins = 1 << nbits
            clear(nbins)
            scan(off, n, nbins, shift, prefshift, prefix)
            bstar, krem = merge_and_find(nbins, krem)
            prefix = (prefix << nbits) | bstar
        return prefix ^ jnp.int32(_INT_MIN)

    T = _TOTAL

    @pl.when(w < 20)
    def _():
        row = jnp.where(w < 10, w, w - 10)
        segoff = jnp.where(w < 10, jnp.int32(_SEG_START[2]),
                           jnp.int32(_SEG_START[4]))
        thrv[pl.ds(0, 16)] = jnp.broadcast_to(
            run_task(row * T + segoff, 160000, 80000), (16,))

    @pl.when(w >= 20)
    def _():
        r = w - 20
        t0 = run_task(r * T + _SEG_START[0], 51200, 25600)
        t1 = run_task(r * T + _SEG_START[6], 25600, 12800)
        thrv[pl.ds(0, 16)] = jnp.where(lane == 1, t1, t0)

    pltpu.sync_copy(thrv, thr_hbm.at[w])


def _sc_thresholds(u_flat):
    mesh = plsc.VectorSubcoreMesh(core_axis_name="c", subcore_axis_name="s")
    f = pl.kernel(
        _sc_body,
        mesh=mesh,
        out_type=jax.ShapeDtypeStruct((32, 16), jnp.int32),
        scratch_types=[
            pltpu.VMEM((_WINW,), jnp.int32),
            pltpu.VMEM((_WINW,), jnp.int32),
            pltpu.VMEM((16 * 4096,), jnp.int32),
            pltpu.VMEM((4096,), jnp.int32),
            pltpu.VMEM((16,), jnp.int32),
            pltpu.SemaphoreType.DMA,
            pltpu.SemaphoreType.DMA,
        ],
        compiler_params=pltpu.CompilerParams(needs_layout_passes=False),
    )
    return f(u_flat)


def _mask_body(u_ref, thr_ref, *out_refs):
    for t, j in enumerate(_BIG):
        si, ea = _SEG_START[j], _SEG_EA[j]
        s = u_ref[:, si:si + ea] ^ jnp.int32(_INT_MIN)
        out_refs[j][...] = (s >= thr_ref[:, t:t + 1]).astype(jnp.float32)
    for j in _SMALL:
        si, ea = _SEG_START[j], _SEG_EA[j]
        k = ea // 2
        s = u_ref[:, si:si + ea] ^ jnp.int32(_INT_MIN)
        cnt0 = jnp.sum((s >= 0).astype(jnp.int32), axis=1, keepdims=True)
        t0 = jnp.where(cnt0 >= k, jnp.int32(0), jnp.int32(_INT_MIN))

        def body(t, thr, s=s, k=k):
            cand = thr | (jnp.int32(1) << (30 - t))
            cnt = jnp.sum((s >= cand).astype(jnp.int32), axis=1, keepdims=True)
            return jnp.where(cnt >= k, cand, thr)

        thr = lax.fori_loop(0, 31, body, t0)
        out_refs[j][...] = (s >= thr).astype(jnp.float32)


def _mask_call(u, thr4):
    return pl.pallas_call(
        _mask_body,
        out_shape=[jax.ShapeDtypeStruct((_B, ea), jnp.float32) for ea in _SEG_EA],
        compiler_params=pltpu.CompilerParams(vmem_limit_bytes=100 * 1024 * 1024),
    )(u, thr4)


def kernel(x, embedding_input, g1_w, g1_b, g2_w, g2_b):
    del x
    u = _mm_call(embedding_input, g1_w, g1_b.reshape(1, 256), g2_w,
                 g2_b.reshape(1, _TOTAL), _gumbel_noise())
    u_flat = jnp.concatenate([u.reshape(-1), jnp.zeros((_WINW,), jnp.int32)])
    thr = _sc_thresholds(u_flat)
    thr4 = jnp.stack([thr[20:30, 0], thr[0:10, 0], thr[10:20, 0],
                      thr[20:30, 1]], axis=1)
    ms = _mask_call(u, thr4)
    return tuple(m.reshape(shp) for m, shp in zip(ms, _SEG_SHAPES))
